# Initial kernel scaffold; baseline (speedup 1.0000x reference)
import jax
import jax.numpy as jnp
from jax.experimental import pallas as pl


def kernel(x, edge_index, batch, edge_attr, nn1_W1, nn1_b1, nn1_W2, nn1_b2, root1, bias1, bn1_w, bn1_b, nn2_W1, nn2_b1, nn2_W2, nn2_b2, root2, bias2, bn2_w, bn2_b, r1_W, r1_b, r2_W, r2_b):
    raise NotImplementedError("write your pallas kernel here")


# speedup vs baseline: 1.7181x; 1.7181x over previous
import functools

import jax
import jax.numpy as jnp
from jax import lax
from jax.experimental import pallas as pl
from jax.experimental.pallas import tpu as pltpu
from jax.experimental.pallas import tpu_sc as plsc

N_NODES = 10000
N_EDGES = 160000
IN_F = 128
HID_F = 16
EDIM_F = 4
NGRAPH = 64

NC = 2
NS = 16
CH = 128
NCHUNK = 40
EDGES_PER_TILE = CH * NCHUNK
E_PAD = NC * NS * EDGES_PER_TILE
N_PAD = 10240
ROWS_PER_TILE = N_PAD // NS
YW = (HID_F + 1) * HID_F
EW = 32

BLK_E = 8192


def _edge_mlp_body(ea_ref, w11_ref, b11_ref, w21_ref, b21_ref, o1_ref, o2_ref):
    j = pl.program_id(0)
    ea = ea_ref[...]
    rows = lax.broadcasted_iota(jnp.int32, (BLK_E, HID_F), 0) + j * BLK_E
    mask = rows < N_EDGES

    def mlp(w_ref, b_ref):
        e = jnp.dot(ea, w_ref[...], preferred_element_type=jnp.float32)
        e = jnp.maximum(e + b_ref[...], 0.0)
        return jnp.where(mask, e, 0.0)

    cw = jnp.where(
        (lax.broadcasted_iota(jnp.int32, (BLK_E, HID_F), 1) == 0) & mask,
        1.0, 0.0)
    o1_ref[:, 0:HID_F] = mlp(w11_ref, b11_ref)
    o1_ref[:, HID_F:EW] = cw
    o2_ref[:, 0:HID_F] = mlp(w21_ref, b21_ref)
    o2_ref[:, HID_F:EW] = cw


def _edge_mlp(ea_pad, w11, b11, w21, b21):
    grid = E_PAD // BLK_E
    return pl.pallas_call(
        _edge_mlp_body,
        grid=(grid,),
        in_specs=[
            pl.BlockSpec((BLK_E, EDIM_F), lambda j: (j, 0)),
            pl.BlockSpec((EDIM_F, HID_F), lambda j: (0, 0)),
            pl.BlockSpec((1, HID_F), lambda j: (0, 0)),
            pl.BlockSpec((EDIM_F, HID_F), lambda j: (0, 0)),
            pl.BlockSpec((1, HID_F), lambda j: (0, 0)),
        ],
        out_specs=[
            pl.BlockSpec((BLK_E, EW), lambda j: (j, 0)),
            pl.BlockSpec((BLK_E, EW), lambda j: (j, 0)),
        ],
        out_shape=[
            jax.ShapeDtypeStruct((E_PAD, EW), jnp.float32),
            jax.ShapeDtypeStruct((E_PAD, EW), jnp.float32),
        ],
    )(ea_pad, w11, b11, w21, b21)


def _matmul_body(a_ref, b_ref, o_ref):
    o_ref[...] = jnp.dot(a_ref[...], b_ref[...],
                         preferred_element_type=jnp.float32)


def _matmul(a, b):
    m, k = a.shape
    _, n = b.shape
    return pl.pallas_call(
        _matmul_body,
        out_shape=jax.ShapeDtypeStruct((m, n), jnp.float32),
    )(a, b)


def _combine1_body(acc_ref, x_ref, root_ref, bias_ref, bnw_ref, bnb_ref,
                   w2p_ref, h1_ref, y2_ref):
    s = acc_ref[0, 0:N_NODES, 0:HID_F] + acc_ref[1, 0:N_NODES, 0:HID_F]
    cnt = (acc_ref[0, 0:N_NODES, HID_F:HID_F + 1]
           + acc_ref[1, 0:N_NODES, HID_F:HID_F + 1])
    aggr = s / jnp.maximum(cnt, 1.0)
    h = aggr + jnp.dot(x_ref[...], root_ref[...],
                       preferred_element_type=jnp.float32) + bias_ref[...]
    mu = jnp.mean(h, axis=0, keepdims=True)
    var = jnp.mean((h - mu) ** 2, axis=0, keepdims=True)
    hn = (h - mu) / jnp.sqrt(var + 1e-5) * bnw_ref[...] + bnb_ref[...]
    h1 = jnp.maximum(hn, 0.0)
    h1_ref[...] = h1
    y2_ref[...] = jnp.dot(h1, w2p_ref[...], preferred_element_type=jnp.float32)


def _combine1(acc, x, root, bias, bnw, bnb, w2p):
    return pl.pallas_call(
        _combine1_body,
        out_shape=[
            jax.ShapeDtypeStruct((N_NODES, HID_F), jnp.float32),
            jax.ShapeDtypeStruct((N_NODES, YW), jnp.float32),
        ],
    )(acc, x, root, bias, bnw, bnb, w2p)


def _combine2_body(acc_ref, h1_ref, root_ref, bias_ref, bnw_ref, bnb_ref,
                   batch_ref, r1w_ref, r1b_ref, r2w_ref, r2b_ref, o_ref):
    s = acc_ref[0, 0:N_NODES, 0:HID_F] + acc_ref[1, 0:N_NODES, 0:HID_F]
    cnt = (acc_ref[0, 0:N_NODES, HID_F:HID_F + 1]
           + acc_ref[1, 0:N_NODES, HID_F:HID_F + 1])
    aggr = s / jnp.maximum(cnt, 1.0)
    h = aggr + jnp.dot(h1_ref[...], root_ref[...],
                       preferred_element_type=jnp.float32) + bias_ref[...]
    mu = jnp.mean(h, axis=0, keepdims=True)
    var = jnp.mean((h - mu) ** 2, axis=0, keepdims=True)
    hn = (h - mu) / jnp.sqrt(var + 1e-5) * bnw_ref[...] + bnb_ref[...]
    h2 = jnp.maximum(hn, 0.0)
    oh = (lax.broadcasted_iota(jnp.int32, (NGRAPH, N_NODES), 0)
          == batch_ref[...]).astype(jnp.float32)
    gs = jnp.dot(oh, h2, preferred_element_type=jnp.float32)
    gc = jnp.sum(oh, axis=1, keepdims=True)
    g = gs / jnp.maximum(gc, 1.0)
    hr = jnp.maximum(
        jnp.dot(g, r1w_ref[...], preferred_element_type=jnp.float32)
        + r1b_ref[...], 0.0)
    o_ref[...] = (jnp.dot(hr, r2w_ref[...], preferred_element_type=jnp.float32)
                  + r2b_ref[...])


def _combine2(acc, h1, root, bias, bnw, bnb, batch2d, r1w, r1b, r2w, r2b):
    return pl.pallas_call(
        _combine2_body,
        out_shape=jax.ShapeDtypeStruct((NGRAPH, 1), jnp.float32),
    )(acc, h1, root, bias, bnw, bnb, batch2d, r1w, r1b, r2w, r2b)


def _sc_body(y_hbm, e_hbm, src_hbm, dst_hbm, out_hbm,
             src_v, dst_v, ybuf, ebuf, obuf, acc, sem):
    cid = lax.axis_index("c")
    sid = lax.axis_index("s")
    tile = cid * NS + sid
    base_edge = tile * EDGES_PER_TILE
    row0 = sid * ROWS_PER_TILE

    def zrow(i, carry):
        obuf[i, 0:16] = jnp.zeros((16,), jnp.float32)
        obuf[i, 16:32] = jnp.zeros((16,), jnp.float32)
        return carry

    lax.fori_loop(0, CH, zrow, 0)
    for k in range(ROWS_PER_TILE // CH):
        pltpu.sync_copy(obuf, acc.at[pl.ds(row0 + k * CH, CH)])
    plsc.subcore_barrier()

    dnums = lax.GatherDimensionNumbers(
        offset_dims=(), collapsed_slice_dims=(0,), start_index_map=(0,))

    def bcast_lane(vec, lane):
        idx = jnp.full((16, 1), lane, jnp.int32)
        return lax.gather(vec, idx, dnums, (1,),
                          mode=lax.GatherScatterMode.PROMISE_IN_BOUNDS)

    def chunk(j, carry):
        eb = base_edge + j * CH
        pltpu.sync_copy(src_hbm.at[pl.ds(eb, CH)], src_v)
        gcp = pltpu.async_copy(y_hbm.at[src_v], ybuf, sem)
        pltpu.sync_copy(dst_hbm.at[pl.ds(eb, CH)], dst_v)
        pltpu.sync_copy(e_hbm.at[pl.ds(eb, CH)], ebuf)
        gcp.wait()

        def edge(c, carry2):
            e_lo = ebuf[c, 0:16]
            e_hi = ebuf[c, 16:32]
            wb = bcast_lane(e_hi, 0)
            m = wb * ybuf[c, 256:272]
            for h in range(HID_F):
                wh = bcast_lane(e_lo, h)
                m = m + wh * ybuf[c, h * 16:(h + 1) * 16]
            obuf[c, 0:16] = m
            obuf[c, 16:32] = e_hi
            return carry2

        lax.fori_loop(0, CH, edge, 0)
        pltpu.sync_copy(obuf, acc.at[dst_v], add=True)
        return carry

    lax.fori_loop(0, NCHUNK, chunk, 0)
    plsc.subcore_barrier()
    pltpu.sync_copy(acc.at[pl.ds(row0, ROWS_PER_TILE)],
                    out_hbm.at[cid, pl.ds(row0, ROWS_PER_TILE)])


def _sc_aggregate(y, eext, src_p, dst_p):
    mesh = plsc.VectorSubcoreMesh(core_axis_name="c", subcore_axis_name="s",
                                  num_cores=NC, num_subcores=NS)
    f = functools.partial(
        pl.kernel,
        out_type=jax.ShapeDtypeStruct((NC, N_PAD, EW), jnp.float32),
        mesh=mesh,
        scratch_types=[
            pltpu.VMEM((CH,), jnp.int32),
            pltpu.VMEM((CH,), jnp.int32),
            pltpu.VMEM((CH, YW), jnp.float32),
            pltpu.VMEM((CH, EW), jnp.float32),
            pltpu.VMEM((CH, EW), jnp.float32),
            pltpu.VMEM_SHARED((N_PAD, EW), jnp.float32),
            pltpu.SemaphoreType.DMA,
        ],
        compiler_params=pltpu.CompilerParams(use_tc_tiling_on_sc=False),
    )(_sc_body)
    return f(y, eext, src_p, dst_p)


def kernel(x, edge_index, batch, edge_attr,
           nn1_W1, nn1_b1, nn1_W2, nn1_b2, root1, bias1, bn1_w, bn1_b,
           nn2_W1, nn2_b1, nn2_W2, nn2_b2, root2, bias2, bn2_w, bn2_b,
           r1_W, r1_b, r2_W, r2_b):
    f32 = jnp.float32
    pad = E_PAD - N_EDGES
    src_p = jnp.concatenate([edge_index[0], jnp.zeros((pad,), jnp.int32)])
    dst_p = jnp.concatenate([edge_index[1], jnp.zeros((pad,), jnp.int32)])
    ea_p = jnp.concatenate([edge_attr, jnp.zeros((pad, EDIM_F), f32)], axis=0)

    w2p1 = jnp.concatenate(
        [nn1_W2.reshape(HID_F, IN_F, HID_F).transpose(1, 0, 2)
         .reshape(IN_F, HID_F * HID_F),
         nn1_b2.reshape(IN_F, HID_F)], axis=1)
    w2p2 = jnp.concatenate(
        [nn2_W2.reshape(HID_F, HID_F, HID_F).transpose(1, 0, 2)
         .reshape(HID_F, HID_F * HID_F),
         nn2_b2.reshape(HID_F, HID_F)], axis=1)

    e1e, e2e = _edge_mlp(ea_p, nn1_W1, nn1_b1.reshape(1, -1),
                         nn2_W1, nn2_b1.reshape(1, -1))
    y1 = _matmul(x, w2p1)
    acc1 = _sc_aggregate(y1, e1e, src_p, dst_p)
    h1, y2 = _combine1(acc1, x, root1, bias1.reshape(1, -1),
                       bn1_w.reshape(1, -1), bn1_b.reshape(1, -1), w2p2)
    acc2 = _sc_aggregate(y2, e2e, src_p, dst_p)
    return _combine2(acc2, h1, root2, bias2.reshape(1, -1),
                     bn2_w.reshape(1, -1), bn2_b.reshape(1, -1),
                     batch.reshape(1, -1), r1_W, r1_b.reshape(1, -1),
                     r2_W, r2_b.reshape(1, -1))


# speedup vs baseline: 2.1337x; 1.2419x over previous
import functools

import jax
import jax.numpy as jnp
from jax import lax
from jax.experimental import pallas as pl
from jax.experimental.pallas import tpu as pltpu
from jax.experimental.pallas import tpu_sc as plsc

N_NODES = 10000
N_EDGES = 160000
IN_F = 128
HID_F = 16
EDIM_F = 4
NGRAPH = 64

NC = 2
NS = 16
CH = 128
NCHUNK = 40
EDGES_PER_TILE = CH * NCHUNK
E_PAD = NC * NS * EDGES_PER_TILE
N_PAD = 10240
ROWS_PER_TILE = N_PAD // NS
YW = (HID_F + 1) * HID_F
EW = 32

BLK_E = 8192


def _edge_mlp_body(ea_ref, w11_ref, b11_ref, w21_ref, b21_ref, o1_ref, o2_ref):
    j = pl.program_id(0)
    ea = ea_ref[...]
    rows = lax.broadcasted_iota(jnp.int32, (BLK_E, HID_F), 0) + j * BLK_E
    mask = rows < N_EDGES

    def mlp(w_ref, b_ref):
        e = jnp.dot(ea, w_ref[...], preferred_element_type=jnp.float32)
        e = jnp.maximum(e + b_ref[...], 0.0)
        return jnp.where(mask, e, 0.0)

    cw = jnp.where(
        (lax.broadcasted_iota(jnp.int32, (BLK_E, HID_F), 1) == 0) & mask,
        1.0, 0.0)
    o1_ref[:, 0:HID_F] = mlp(w11_ref, b11_ref)
    o1_ref[:, HID_F:EW] = cw
    o2_ref[:, 0:HID_F] = mlp(w21_ref, b21_ref)
    o2_ref[:, HID_F:EW] = cw


def _edge_mlp(ea_pad, w11, b11, w21, b21):
    grid = E_PAD // BLK_E
    return pl.pallas_call(
        _edge_mlp_body,
        grid=(grid,),
        in_specs=[
            pl.BlockSpec((BLK_E, EDIM_F), lambda j: (j, 0)),
            pl.BlockSpec((EDIM_F, HID_F), lambda j: (0, 0)),
            pl.BlockSpec((1, HID_F), lambda j: (0, 0)),
            pl.BlockSpec((EDIM_F, HID_F), lambda j: (0, 0)),
            pl.BlockSpec((1, HID_F), lambda j: (0, 0)),
        ],
        out_specs=[
            pl.BlockSpec((BLK_E, EW), lambda j: (j, 0)),
            pl.BlockSpec((BLK_E, EW), lambda j: (j, 0)),
        ],
        out_shape=[
            jax.ShapeDtypeStruct((E_PAD, EW), jnp.float32),
            jax.ShapeDtypeStruct((E_PAD, EW), jnp.float32),
        ],
    )(ea_pad, w11, b11, w21, b21)


def _matmul_body(a_ref, b_ref, o_ref):
    o_ref[...] = jnp.dot(a_ref[...], b_ref[...],
                         preferred_element_type=jnp.float32)


def _matmul(a, b):
    m, k = a.shape
    _, n = b.shape
    return pl.pallas_call(
        _matmul_body,
        out_shape=jax.ShapeDtypeStruct((m, n), jnp.float32),
    )(a, b)


def _combine1_body(acc_ref, x_ref, root_ref, bias_ref, bnw_ref, bnb_ref,
                   w2p_ref, h1_ref, y2_ref):
    s = acc_ref[0, 0:N_NODES, 0:HID_F] + acc_ref[1, 0:N_NODES, 0:HID_F]
    cnt = (acc_ref[0, 0:N_NODES, HID_F:HID_F + 1]
           + acc_ref[1, 0:N_NODES, HID_F:HID_F + 1])
    aggr = s / jnp.maximum(cnt, 1.0)
    h = aggr + jnp.dot(x_ref[...], root_ref[...],
                       preferred_element_type=jnp.float32) + bias_ref[...]
    mu = jnp.mean(h, axis=0, keepdims=True)
    var = jnp.mean((h - mu) ** 2, axis=0, keepdims=True)
    hn = (h - mu) / jnp.sqrt(var + 1e-5) * bnw_ref[...] + bnb_ref[...]
    h1 = jnp.maximum(hn, 0.0)
    h1_ref[...] = h1
    y2_ref[...] = jnp.dot(h1, w2p_ref[...], preferred_element_type=jnp.float32)


def _combine1(acc, x, root, bias, bnw, bnb, w2p):
    return pl.pallas_call(
        _combine1_body,
        out_shape=[
            jax.ShapeDtypeStruct((N_NODES, HID_F), jnp.float32),
            jax.ShapeDtypeStruct((N_NODES, YW), jnp.float32),
        ],
    )(acc, x, root, bias, bnw, bnb, w2p)


def _combine2_body(acc_ref, h1_ref, root_ref, bias_ref, bnw_ref, bnb_ref,
                   batch_ref, r1w_ref, r1b_ref, r2w_ref, r2b_ref, o_ref):
    s = acc_ref[0, 0:N_NODES, 0:HID_F] + acc_ref[1, 0:N_NODES, 0:HID_F]
    cnt = (acc_ref[0, 0:N_NODES, HID_F:HID_F + 1]
           + acc_ref[1, 0:N_NODES, HID_F:HID_F + 1])
    aggr = s / jnp.maximum(cnt, 1.0)
    h = aggr + jnp.dot(h1_ref[...], root_ref[...],
                       preferred_element_type=jnp.float32) + bias_ref[...]
    mu = jnp.mean(h, axis=0, keepdims=True)
    var = jnp.mean((h - mu) ** 2, axis=0, keepdims=True)
    hn = (h - mu) / jnp.sqrt(var + 1e-5) * bnw_ref[...] + bnb_ref[...]
    h2 = jnp.maximum(hn, 0.0)
    oh = (lax.broadcasted_iota(jnp.int32, (NGRAPH, N_NODES), 0)
          == batch_ref[...]).astype(jnp.float32)
    gs = jnp.dot(oh, h2, preferred_element_type=jnp.float32)
    gc = jnp.sum(oh, axis=1, keepdims=True)
    g = gs / jnp.maximum(gc, 1.0)
    hr = jnp.maximum(
        jnp.dot(g, r1w_ref[...], preferred_element_type=jnp.float32)
        + r1b_ref[...], 0.0)
    o_ref[...] = (jnp.dot(hr, r2w_ref[...], preferred_element_type=jnp.float32)
                  + r2b_ref[...])


def _combine2(acc, h1, root, bias, bnw, bnb, batch2d, r1w, r1b, r2w, r2b):
    return pl.pallas_call(
        _combine2_body,
        out_shape=jax.ShapeDtypeStruct((NGRAPH, 1), jnp.float32),
    )(acc, h1, root, bias, bnw, bnb, batch2d, r1w, r1b, r2w, r2b)


def _sc_body(y_hbm, e_hbm, src_hbm, dst_hbm, out_hbm,
             src_v, dst_v, ybuf, ebuf, obuf, acc,
             semy0, semy1, seme0, seme1):
    cid = lax.axis_index("c")
    sid = lax.axis_index("s")
    tile = cid * NS + sid
    base_chunk = tile * NCHUNK
    row0 = sid * ROWS_PER_TILE
    semy = (semy0, semy1)
    seme = (seme0, seme1)

    pltpu.sync_copy(src_hbm.at[pl.ds(base_chunk, NCHUNK)], src_v)
    pltpu.sync_copy(dst_hbm.at[pl.ds(base_chunk, NCHUNK)], dst_v)

    def zrow(i, carry):
        obuf[i, 0:16] = jnp.zeros((16,), jnp.float32)
        obuf[i, 16:32] = jnp.zeros((16,), jnp.float32)
        return carry

    lax.fori_loop(0, CH, zrow, 0)
    for k in range(ROWS_PER_TILE // CH):
        pltpu.sync_copy(obuf, acc.at[pl.ds(row0 + k * CH, CH)])
    plsc.subcore_barrier()

    dnums = lax.GatherDimensionNumbers(
        offset_dims=(), collapsed_slice_dims=(0,), start_index_map=(0,))

    def bcast_lane(vec, lane):
        idx = jnp.full((16, 1), lane, jnp.int32)
        return lax.gather(vec, idx, dnums, (1,),
                          mode=lax.GatherScatterMode.PROMISE_IN_BOUNDS)

    def issue(j, b):
        pltpu.async_copy(y_hbm.at[src_v.at[j]], ybuf.at[b], semy[b])
        pltpu.async_copy(e_hbm.at[pl.ds((base_chunk + j) * CH, CH)],
                         ebuf.at[b], seme[b])

    for b in range(2):
        issue(b, b)

    def chunk2(j2, carry):
        for b in range(2):
            j = j2 * 2 + b
            pltpu.make_async_copy(y_hbm.at[src_v.at[0]], ybuf.at[b],
                                  semy[b]).wait()
            pltpu.make_async_copy(e_hbm.at[pl.ds(0, CH)], ebuf.at[b],
                                  seme[b]).wait()

            def edge(c, carry2):
                e_lo = ebuf[b, c, 0:16]
                e_hi = ebuf[b, c, 16:32]
                wb = bcast_lane(e_hi, 0)
                m = wb * ybuf[b, c, 256:272]
                for h in range(HID_F):
                    wh = bcast_lane(e_lo, h)
                    m = m + wh * ybuf[b, c, h * 16:(h + 1) * 16]
                obuf[c, 0:16] = m
                obuf[c, 16:32] = e_hi
                return carry2

            lax.fori_loop(0, CH, edge, 0)
            pltpu.sync_copy(obuf, acc.at[dst_v.at[j]], add=True)
            issue(jnp.minimum(j + 2, NCHUNK - 1), b)
        return carry

    lax.fori_loop(0, NCHUNK // 2, chunk2, 0)
    for b in range(2):
        pltpu.make_async_copy(y_hbm.at[src_v.at[0]], ybuf.at[b],
                              semy[b]).wait()
        pltpu.make_async_copy(e_hbm.at[pl.ds(0, CH)], ebuf.at[b],
                              seme[b]).wait()
    plsc.subcore_barrier()
    pltpu.sync_copy(acc.at[pl.ds(row0, ROWS_PER_TILE)],
                    out_hbm.at[cid, pl.ds(row0, ROWS_PER_TILE)])


def _sc_aggregate(y, eext, src_2d, dst_2d):
    mesh = plsc.VectorSubcoreMesh(core_axis_name="c", subcore_axis_name="s",
                                  num_cores=NC, num_subcores=NS)
    f = functools.partial(
        pl.kernel,
        out_type=jax.ShapeDtypeStruct((NC, N_PAD, EW), jnp.float32),
        mesh=mesh,
        scratch_types=[
            pltpu.VMEM((NCHUNK, CH), jnp.int32),
            pltpu.VMEM((NCHUNK, CH), jnp.int32),
            pltpu.VMEM((2, CH, YW), jnp.float32),
            pltpu.VMEM((2, CH, EW), jnp.float32),
            pltpu.VMEM((CH, EW), jnp.float32),
            pltpu.VMEM_SHARED((N_PAD, EW), jnp.float32),
            pltpu.SemaphoreType.DMA,
            pltpu.SemaphoreType.DMA,
            pltpu.SemaphoreType.DMA,
            pltpu.SemaphoreType.DMA,
        ],
        compiler_params=pltpu.CompilerParams(use_tc_tiling_on_sc=False),
    )(_sc_body)
    return f(y, eext, src_2d, dst_2d)


def kernel(x, edge_index, batch, edge_attr,
           nn1_W1, nn1_b1, nn1_W2, nn1_b2, root1, bias1, bn1_w, bn1_b,
           nn2_W1, nn2_b1, nn2_W2, nn2_b2, root2, bias2, bn2_w, bn2_b,
           r1_W, r1_b, r2_W, r2_b):
    f32 = jnp.float32
    pad = E_PAD - N_EDGES
    src_p = jnp.concatenate([edge_index[0], jnp.zeros((pad,), jnp.int32)])
    dst_p = jnp.concatenate([edge_index[1], jnp.zeros((pad,), jnp.int32)])
    ea_p = jnp.concatenate([edge_attr, jnp.zeros((pad, EDIM_F), f32)], axis=0)

    w2p1 = jnp.concatenate(
        [nn1_W2.reshape(HID_F, IN_F, HID_F).transpose(1, 0, 2)
         .reshape(IN_F, HID_F * HID_F),
         nn1_b2.reshape(IN_F, HID_F)], axis=1)
    w2p2 = jnp.concatenate(
        [nn2_W2.reshape(HID_F, HID_F, HID_F).transpose(1, 0, 2)
         .reshape(HID_F, HID_F * HID_F),
         nn2_b2.reshape(HID_F, HID_F)], axis=1)

    src_2d = src_p.reshape(E_PAD // CH, CH)
    dst_2d = dst_p.reshape(E_PAD // CH, CH)

    e1e, e2e = _edge_mlp(ea_p, nn1_W1, nn1_b1.reshape(1, -1),
                         nn2_W1, nn2_b1.reshape(1, -1))
    y1 = _matmul(x, w2p1)
    acc1 = _sc_aggregate(y1, e1e, src_2d, dst_2d)
    h1, y2 = _combine1(acc1, x, root1, bias1.reshape(1, -1),
                       bn1_w.reshape(1, -1), bn1_b.reshape(1, -1), w2p2)
    acc2 = _sc_aggregate(y2, e2e, src_2d, dst_2d)
    return _combine2(acc2, h1, root2, bias2.reshape(1, -1),
                     bn2_w.reshape(1, -1), bn2_b.reshape(1, -1),
                     batch.reshape(1, -1), r1_W, r1_b.reshape(1, -1),
                     r2_W, r2_b.reshape(1, -1))


# speedup vs baseline: 2.1768x; 1.0202x over previous
import functools

import jax
import jax.numpy as jnp
from jax import lax
from jax.experimental import pallas as pl
from jax.experimental.pallas import tpu as pltpu
from jax.experimental.pallas import tpu_sc as plsc

N_NODES = 10000
N_EDGES = 160000
IN_F = 128
HID_F = 16
EDIM_F = 4
NGRAPH = 64

NC = 2
NS = 16
CH = 128
NCHUNK = 40
C0 = 52
C1 = 28
EDGES_PER_TILE = CH * NCHUNK
E_PAD = NC * NS * EDGES_PER_TILE
NCH_ROWS = E_PAD // CH
N_PAD = 10240
ROWS_PER_TILE = N_PAD // NS
YW = (HID_F + 1) * HID_F
EW = 32

BLK_E = 8192


def _edge_mlp_body(ea_ref, w11_ref, b11_ref, w21_ref, b21_ref, o1_ref, o2_ref):
    j = pl.program_id(0)
    ea = ea_ref[...]
    rows = lax.broadcasted_iota(jnp.int32, (BLK_E, HID_F), 0) + j * BLK_E
    mask = rows < N_EDGES

    def mlp(w_ref, b_ref):
        e = jnp.dot(ea, w_ref[...], preferred_element_type=jnp.float32)
        e = jnp.maximum(e + b_ref[...], 0.0)
        return jnp.where(mask, e, 0.0)

    cw = jnp.where(
        (lax.broadcasted_iota(jnp.int32, (BLK_E, HID_F), 1) == 0) & mask,
        1.0, 0.0)
    o1_ref[:, 0:HID_F] = mlp(w11_ref, b11_ref)
    o1_ref[:, HID_F:EW] = cw
    o2_ref[:, 0:HID_F] = mlp(w21_ref, b21_ref)
    o2_ref[:, HID_F:EW] = cw


def _edge_mlp(ea_pad, w11, b11, w21, b21):
    grid = E_PAD // BLK_E
    return pl.pallas_call(
        _edge_mlp_body,
        grid=(grid,),
        in_specs=[
            pl.BlockSpec((BLK_E, EDIM_F), lambda j: (j, 0)),
            pl.BlockSpec((EDIM_F, HID_F), lambda j: (0, 0)),
            pl.BlockSpec((1, HID_F), lambda j: (0, 0)),
            pl.BlockSpec((EDIM_F, HID_F), lambda j: (0, 0)),
            pl.BlockSpec((1, HID_F), lambda j: (0, 0)),
        ],
        out_specs=[
            pl.BlockSpec((BLK_E, EW), lambda j: (j, 0)),
            pl.BlockSpec((BLK_E, EW), lambda j: (j, 0)),
        ],
        out_shape=[
            jax.ShapeDtypeStruct((E_PAD, EW), jnp.float32),
            jax.ShapeDtypeStruct((E_PAD, EW), jnp.float32),
        ],
    )(ea_pad, w11, b11, w21, b21)


def _matmul_body(a_ref, b_ref, o_ref):
    o_ref[...] = jnp.dot(a_ref[...], b_ref[...],
                         preferred_element_type=jnp.float32)


def _matmul(a, b):
    m, k = a.shape
    _, n = b.shape
    return pl.pallas_call(
        _matmul_body,
        out_shape=jax.ShapeDtypeStruct((m, n), jnp.float32),
    )(a, b)


def _combine1_body(acc_ref, x_ref, root_ref, bias_ref, bnw_ref, bnb_ref,
                   w2p_ref, h1_ref, y2_ref):
    s = acc_ref[0, 0:N_NODES, 0:HID_F] + acc_ref[1, 0:N_NODES, 0:HID_F]
    cnt = (acc_ref[0, 0:N_NODES, HID_F:HID_F + 1]
           + acc_ref[1, 0:N_NODES, HID_F:HID_F + 1])
    aggr = s / jnp.maximum(cnt, 1.0)
    h = aggr + jnp.dot(x_ref[...], root_ref[...],
                       preferred_element_type=jnp.float32) + bias_ref[...]
    mu = jnp.mean(h, axis=0, keepdims=True)
    var = jnp.mean((h - mu) ** 2, axis=0, keepdims=True)
    hn = (h - mu) / jnp.sqrt(var + 1e-5) * bnw_ref[...] + bnb_ref[...]
    h1 = jnp.maximum(hn, 0.0)
    h1_ref[...] = h1
    y2_ref[...] = jnp.dot(h1, w2p_ref[...], preferred_element_type=jnp.float32)


def _combine1(acc, x, root, bias, bnw, bnb, w2p):
    return pl.pallas_call(
        _combine1_body,
        out_shape=[
            jax.ShapeDtypeStruct((N_NODES, HID_F), jnp.float32),
            jax.ShapeDtypeStruct((N_NODES, YW), jnp.float32),
        ],
    )(acc, x, root, bias, bnw, bnb, w2p)


def _combine2_body(acc_ref, h1_ref, root_ref, bias_ref, bnw_ref, bnb_ref,
                   batch_ref, r1w_ref, r1b_ref, r2w_ref, r2b_ref, o_ref):
    s = acc_ref[0, 0:N_NODES, 0:HID_F] + acc_ref[1, 0:N_NODES, 0:HID_F]
    cnt = (acc_ref[0, 0:N_NODES, HID_F:HID_F + 1]
           + acc_ref[1, 0:N_NODES, HID_F:HID_F + 1])
    aggr = s / jnp.maximum(cnt, 1.0)
    h = aggr + jnp.dot(h1_ref[...], root_ref[...],
                       preferred_element_type=jnp.float32) + bias_ref[...]
    mu = jnp.mean(h, axis=0, keepdims=True)
    var = jnp.mean((h - mu) ** 2, axis=0, keepdims=True)
    hn = (h - mu) / jnp.sqrt(var + 1e-5) * bnw_ref[...] + bnb_ref[...]
    h2 = jnp.maximum(hn, 0.0)
    oh = (lax.broadcasted_iota(jnp.int32, (NGRAPH, N_NODES), 0)
          == batch_ref[...]).astype(jnp.float32)
    gs = jnp.dot(oh, h2, preferred_element_type=jnp.float32)
    gc = jnp.sum(oh, axis=1, keepdims=True)
    g = gs / jnp.maximum(gc, 1.0)
    hr = jnp.maximum(
        jnp.dot(g, r1w_ref[...], preferred_element_type=jnp.float32)
        + r1b_ref[...], 0.0)
    o_ref[...] = (jnp.dot(hr, r2w_ref[...], preferred_element_type=jnp.float32)
                  + r2b_ref[...])


def _combine2(acc, h1, root, bias, bnw, bnb, batch2d, r1w, r1b, r2w, r2b):
    return pl.pallas_call(
        _combine2_body,
        out_shape=jax.ShapeDtypeStruct((NGRAPH, 1), jnp.float32),
    )(acc, h1, root, bias, bnw, bnb, batch2d, r1w, r1b, r2w, r2b)


def _sc_body(y_hbm, e_hbm, src_hbm, dst_hbm, out_hbm,
             src_v, dst_v, ybuf, ebuf, obuf, acc,
             semy0, semy1, seme0, seme1):
    cid = lax.axis_index("c")
    sid = lax.axis_index("s")
    nch = jnp.where(cid == 0, C0, C1)
    base_chunk = cid * (NS * C0) + sid * nch
    row0 = sid * ROWS_PER_TILE
    semy = (semy0, semy1)
    seme = (seme0, seme1)

    stage = jnp.minimum(base_chunk, NCH_ROWS - C0)
    off = base_chunk - stage
    pltpu.sync_copy(src_hbm.at[pl.ds(stage, C0)], src_v)
    pltpu.sync_copy(dst_hbm.at[pl.ds(stage, C0)], dst_v)

    def zrow(i, carry):
        obuf[i, 0:16] = jnp.zeros((16,), jnp.float32)
        obuf[i, 16:32] = jnp.zeros((16,), jnp.float32)
        return carry

    lax.fori_loop(0, CH, zrow, 0)
    for k in range(ROWS_PER_TILE // CH):
        pltpu.sync_copy(obuf, acc.at[pl.ds(row0 + k * CH, CH)])
    plsc.subcore_barrier()

    dnums = lax.GatherDimensionNumbers(
        offset_dims=(), collapsed_slice_dims=(0,), start_index_map=(0,))

    def bcast_lane(vec, lane):
        idx = jnp.full((16, 1), lane, jnp.int32)
        return lax.gather(vec, idx, dnums, (1,),
                          mode=lax.GatherScatterMode.PROMISE_IN_BOUNDS)

    def issue(j, b):
        pltpu.async_copy(y_hbm.at[src_v.at[off + j]], ybuf.at[b], semy[b])
        pltpu.async_copy(e_hbm.at[pl.ds((base_chunk + j) * CH, CH)],
                         ebuf.at[b], seme[b])

    for b in range(2):
        issue(b, b)

    def chunk2(j2, carry):
        for b in range(2):
            j = j2 * 2 + b
            pltpu.make_async_copy(y_hbm.at[src_v.at[0]], ybuf.at[b],
                                  semy[b]).wait()
            pltpu.make_async_copy(e_hbm.at[pl.ds(0, CH)], ebuf.at[b],
                                  seme[b]).wait()

            def edge(c, carry2):
                e_lo = ebuf[b, c, 0:16]
                e_hi = ebuf[b, c, 16:32]
                wb = bcast_lane(e_hi, 0)
                m = wb * ybuf[b, c, 256:272]
                for h in range(HID_F):
                    wh = bcast_lane(e_lo, h)
                    m = m + wh * ybuf[b, c, h * 16:(h + 1) * 16]
                obuf[c, 0:16] = m
                obuf[c, 16:32] = e_hi
                return carry2

            lax.fori_loop(0, CH, edge, 0)
            pltpu.sync_copy(obuf, acc.at[dst_v.at[off + j]], add=True)
            issue(jnp.minimum(j + 2, nch - 1), b)
        return carry

    lax.fori_loop(0, nch // 2, chunk2, 0)
    for b in range(2):
        pltpu.make_async_copy(y_hbm.at[src_v.at[0]], ybuf.at[b],
                              semy[b]).wait()
        pltpu.make_async_copy(e_hbm.at[pl.ds(0, CH)], ebuf.at[b],
                              seme[b]).wait()
    plsc.subcore_barrier()
    pltpu.sync_copy(acc.at[pl.ds(row0, ROWS_PER_TILE)],
                    out_hbm.at[cid, pl.ds(row0, ROWS_PER_TILE)])


def _sc_aggregate(y, eext, src_2d, dst_2d):
    mesh = plsc.VectorSubcoreMesh(core_axis_name="c", subcore_axis_name="s",
                                  num_cores=NC, num_subcores=NS)
    f = functools.partial(
        pl.kernel,
        out_type=jax.ShapeDtypeStruct((NC, N_PAD, EW), jnp.float32),
        mesh=mesh,
        scratch_types=[
            pltpu.VMEM((C0, CH), jnp.int32),
            pltpu.VMEM((C0, CH), jnp.int32),
            pltpu.VMEM((2, CH, YW), jnp.float32),
            pltpu.VMEM((2, CH, EW), jnp.float32),
            pltpu.VMEM((CH, EW), jnp.float32),
            pltpu.VMEM_SHARED((N_PAD, EW), jnp.float32),
            pltpu.SemaphoreType.DMA,
            pltpu.SemaphoreType.DMA,
            pltpu.SemaphoreType.DMA,
            pltpu.SemaphoreType.DMA,
        ],
        compiler_params=pltpu.CompilerParams(use_tc_tiling_on_sc=False),
    )(_sc_body)
    return f(y, eext, src_2d, dst_2d)


def kernel(x, edge_index, batch, edge_attr,
           nn1_W1, nn1_b1, nn1_W2, nn1_b2, root1, bias1, bn1_w, bn1_b,
           nn2_W1, nn2_b1, nn2_W2, nn2_b2, root2, bias2, bn2_w, bn2_b,
           r1_W, r1_b, r2_W, r2_b):
    f32 = jnp.float32
    pad = E_PAD - N_EDGES
    src_p = jnp.concatenate([edge_index[0], jnp.zeros((pad,), jnp.int32)])
    dst_p = jnp.concatenate([edge_index[1], jnp.zeros((pad,), jnp.int32)])
    ea_p = jnp.concatenate([edge_attr, jnp.zeros((pad, EDIM_F), f32)], axis=0)

    w2p1 = jnp.concatenate(
        [nn1_W2.reshape(HID_F, IN_F, HID_F).transpose(1, 0, 2)
         .reshape(IN_F, HID_F * HID_F),
         nn1_b2.reshape(IN_F, HID_F)], axis=1)
    w2p2 = jnp.concatenate(
        [nn2_W2.reshape(HID_F, HID_F, HID_F).transpose(1, 0, 2)
         .reshape(HID_F, HID_F * HID_F),
         nn2_b2.reshape(HID_F, HID_F)], axis=1)

    src_2d = src_p.reshape(E_PAD // CH, CH)
    dst_2d = dst_p.reshape(E_PAD // CH, CH)

    e1e, e2e = _edge_mlp(ea_p, nn1_W1, nn1_b1.reshape(1, -1),
                         nn2_W1, nn2_b1.reshape(1, -1))
    y1 = _matmul(x, w2p1)
    acc1 = _sc_aggregate(y1, e1e, src_2d, dst_2d)
    h1, y2 = _combine1(acc1, x, root1, bias1.reshape(1, -1),
                       bn1_w.reshape(1, -1), bn1_b.reshape(1, -1), w2p2)
    acc2 = _sc_aggregate(y2, e2e, src_2d, dst_2d)
    return _combine2(acc2, h1, root2, bias2.reshape(1, -1),
                     bn2_w.reshape(1, -1), bn2_b.reshape(1, -1),
                     batch.reshape(1, -1), r1_W, r1_b.reshape(1, -1),
                     r2_W, r2_b.reshape(1, -1))


# speedup vs baseline: 2.2005x; 1.0109x over previous
import functools

import jax
import jax.numpy as jnp
from jax import lax
from jax.experimental import pallas as pl
from jax.experimental.pallas import tpu as pltpu
from jax.experimental.pallas import tpu_sc as plsc

N_NODES = 10000
N_EDGES = 160000
IN_F = 128
HID_F = 16
EDIM_F = 4
NGRAPH = 64

NC = 2
NS = 16
CH = 128
NCHUNK = 40
C0 = 52
C1 = 28
EDGES_PER_TILE = CH * NCHUNK
E_PAD = NC * NS * EDGES_PER_TILE
NCH_ROWS = E_PAD // CH
N_PAD = 10240
ROWS_PER_TILE = N_PAD // NS
YW = (HID_F + 1) * HID_F
YWB = 288
EW = 32

BLK_E = 8192


def _edge_mlp_body(ea_ref, w11_ref, b11_ref, w21_ref, b21_ref, o1_ref, o2_ref):
    j = pl.program_id(0)
    ea = ea_ref[...]
    rows = lax.broadcasted_iota(jnp.int32, (BLK_E, HID_F), 0) + j * BLK_E
    mask = rows < N_EDGES

    def mlp(w_ref, b_ref):
        e = jnp.dot(ea, w_ref[...], preferred_element_type=jnp.float32)
        e = jnp.maximum(e + b_ref[...], 0.0)
        return jnp.where(mask, e, 0.0)

    cw = jnp.where(
        (lax.broadcasted_iota(jnp.int32, (BLK_E, HID_F), 1) == 0) & mask,
        1.0, 0.0)
    o1_ref[:, 0:HID_F] = mlp(w11_ref, b11_ref)
    o1_ref[:, HID_F:EW] = cw
    o2_ref[:, 0:HID_F] = mlp(w21_ref, b21_ref)
    o2_ref[:, HID_F:EW] = cw


def _edge_mlp(ea_pad, w11, b11, w21, b21):
    grid = E_PAD // BLK_E
    return pl.pallas_call(
        _edge_mlp_body,
        grid=(grid,),
        in_specs=[
            pl.BlockSpec((BLK_E, EDIM_F), lambda j: (j, 0)),
            pl.BlockSpec((EDIM_F, HID_F), lambda j: (0, 0)),
            pl.BlockSpec((1, HID_F), lambda j: (0, 0)),
            pl.BlockSpec((EDIM_F, HID_F), lambda j: (0, 0)),
            pl.BlockSpec((1, HID_F), lambda j: (0, 0)),
        ],
        out_specs=[
            pl.BlockSpec((BLK_E, EW), lambda j: (j, 0)),
            pl.BlockSpec((BLK_E, EW), lambda j: (j, 0)),
        ],
        out_shape=[
            jax.ShapeDtypeStruct((E_PAD, EW), jnp.float32),
            jax.ShapeDtypeStruct((E_PAD, EW), jnp.float32),
        ],
    )(ea_pad, w11, b11, w21, b21)


def _matmul_body(a_ref, b_ref, o_ref):
    o_ref[...] = jnp.dot(a_ref[...], b_ref[...],
                         preferred_element_type=jnp.float32
                         ).astype(o_ref.dtype)


def _matmul(a, b, out_dtype=jnp.float32):
    m, k = a.shape
    _, n = b.shape
    return pl.pallas_call(
        _matmul_body,
        out_shape=jax.ShapeDtypeStruct((m, n), out_dtype),
    )(a, b)


def _combine1_body(acc_ref, x_ref, root_ref, bias_ref, bnw_ref, bnb_ref,
                   w2p_ref, h1_ref, y2_ref):
    s = acc_ref[0, 0:N_NODES, 0:HID_F] + acc_ref[1, 0:N_NODES, 0:HID_F]
    cnt = (acc_ref[0, 0:N_NODES, HID_F:HID_F + 1]
           + acc_ref[1, 0:N_NODES, HID_F:HID_F + 1])
    aggr = s / jnp.maximum(cnt, 1.0)
    h = aggr + jnp.dot(x_ref[...], root_ref[...],
                       preferred_element_type=jnp.float32) + bias_ref[...]
    mu = jnp.mean(h, axis=0, keepdims=True)
    var = jnp.mean((h - mu) ** 2, axis=0, keepdims=True)
    hn = (h - mu) / jnp.sqrt(var + 1e-5) * bnw_ref[...] + bnb_ref[...]
    h1 = jnp.maximum(hn, 0.0)
    h1_ref[...] = h1
    y2_ref[...] = jnp.dot(h1, w2p_ref[...], preferred_element_type=jnp.float32
                          ).astype(y2_ref.dtype)


def _combine1(acc, x, root, bias, bnw, bnb, w2p):
    return pl.pallas_call(
        _combine1_body,
        out_shape=[
            jax.ShapeDtypeStruct((N_NODES, HID_F), jnp.float32),
            jax.ShapeDtypeStruct((N_NODES, YWB), jnp.bfloat16),
        ],
    )(acc, x, root, bias, bnw, bnb, w2p)


def _combine2_body(acc_ref, h1_ref, root_ref, bias_ref, bnw_ref, bnb_ref,
                   batch_ref, r1w_ref, r1b_ref, r2w_ref, r2b_ref, o_ref):
    s = acc_ref[0, 0:N_NODES, 0:HID_F] + acc_ref[1, 0:N_NODES, 0:HID_F]
    cnt = (acc_ref[0, 0:N_NODES, HID_F:HID_F + 1]
           + acc_ref[1, 0:N_NODES, HID_F:HID_F + 1])
    aggr = s / jnp.maximum(cnt, 1.0)
    h = aggr + jnp.dot(h1_ref[...], root_ref[...],
                       preferred_element_type=jnp.float32) + bias_ref[...]
    mu = jnp.mean(h, axis=0, keepdims=True)
    var = jnp.mean((h - mu) ** 2, axis=0, keepdims=True)
    hn = (h - mu) / jnp.sqrt(var + 1e-5) * bnw_ref[...] + bnb_ref[...]
    h2 = jnp.maximum(hn, 0.0)
    oh = (lax.broadcasted_iota(jnp.int32, (NGRAPH, N_NODES), 0)
          == batch_ref[...]).astype(jnp.float32)
    gs = jnp.dot(oh, h2, preferred_element_type=jnp.float32)
    gc = jnp.sum(oh, axis=1, keepdims=True)
    g = gs / jnp.maximum(gc, 1.0)
    hr = jnp.maximum(
        jnp.dot(g, r1w_ref[...], preferred_element_type=jnp.float32)
        + r1b_ref[...], 0.0)
    o_ref[...] = (jnp.dot(hr, r2w_ref[...], preferred_element_type=jnp.float32)
                  + r2b_ref[...])


def _combine2(acc, h1, root, bias, bnw, bnb, batch2d, r1w, r1b, r2w, r2b):
    return pl.pallas_call(
        _combine2_body,
        out_shape=jax.ShapeDtypeStruct((NGRAPH, 1), jnp.float32),
    )(acc, h1, root, bias, bnw, bnb, batch2d, r1w, r1b, r2w, r2b)


def _sc_body(y_hbm, e_hbm, src_hbm, dst_hbm, out_hbm,
             src_v, dst_v, ybuf, ebuf, obuf, acc,
             semy0, semy1, seme0, seme1):
    cid = lax.axis_index("c")
    sid = lax.axis_index("s")
    nch = jnp.where(cid == 0, C0, C1)
    base_chunk = cid * (NS * C0) + sid * nch
    row0 = sid * ROWS_PER_TILE
    semy = (semy0, semy1)
    seme = (seme0, seme1)

    stage = jnp.minimum(base_chunk, NCH_ROWS - C0)
    off = base_chunk - stage
    pltpu.sync_copy(src_hbm.at[pl.ds(stage, C0)], src_v)
    pltpu.sync_copy(dst_hbm.at[pl.ds(stage, C0)], dst_v)

    def zrow(i, carry):
        obuf[i, 0:16] = jnp.zeros((16,), jnp.float32)
        obuf[i, 16:32] = jnp.zeros((16,), jnp.float32)
        return carry

    lax.fori_loop(0, CH, zrow, 0)
    for k in range(ROWS_PER_TILE // CH):
        pltpu.sync_copy(obuf, acc.at[pl.ds(row0 + k * CH, CH)])
    plsc.subcore_barrier()

    dnums = lax.GatherDimensionNumbers(
        offset_dims=(), collapsed_slice_dims=(0,), start_index_map=(0,))

    def bcast_lane(vec, lane):
        idx = jnp.full((16, 1), lane, jnp.int32)
        return lax.gather(vec, idx, dnums, (1,),
                          mode=lax.GatherScatterMode.PROMISE_IN_BOUNDS)

    def issue(j, b):
        pltpu.async_copy(y_hbm.at[src_v.at[off + j]], ybuf.at[b], semy[b])
        pltpu.async_copy(e_hbm.at[pl.ds((base_chunk + j) * CH, CH)],
                         ebuf.at[b], seme[b])

    for b in range(2):
        issue(b, b)

    def chunk2(j2, carry):
        for b in range(2):
            j = j2 * 2 + b
            pltpu.make_async_copy(y_hbm.at[src_v.at[0]], ybuf.at[b],
                                  semy[b]).wait()
            pltpu.make_async_copy(e_hbm.at[pl.ds(0, CH)], ebuf.at[b],
                                  seme[b]).wait()

            def edge(c, carry2):
                e_lo = ebuf[b, c, 0:16]
                e_hi = ebuf[b, c, 16:32]
                vb = ybuf[b, c, 8 * 32:8 * 32 + 32]
                ba, _ = plsc.unpack(vb, format=plsc.PackFormat.INTERLEAVED)
                m = bcast_lane(e_hi, 0) * ba
                for k in range(8):
                    v = ybuf[b, c, k * 32:(k + 1) * 32]
                    ya, yb = plsc.unpack(v,
                                         format=plsc.PackFormat.INTERLEAVED)
                    m = m + bcast_lane(e_lo, 2 * k) * ya
                    m = m + bcast_lane(e_lo, 2 * k + 1) * yb
                obuf[c, 0:16] = m
                obuf[c, 16:32] = e_hi
                return carry2

            lax.fori_loop(0, CH, edge, 0)
            pltpu.sync_copy(obuf, acc.at[dst_v.at[off + j]], add=True)
            issue(jnp.minimum(j + 2, nch - 1), b)
        return carry

    lax.fori_loop(0, nch // 2, chunk2, 0)
    for b in range(2):
        pltpu.make_async_copy(y_hbm.at[src_v.at[0]], ybuf.at[b],
                              semy[b]).wait()
        pltpu.make_async_copy(e_hbm.at[pl.ds(0, CH)], ebuf.at[b],
                              seme[b]).wait()
    plsc.subcore_barrier()
    pltpu.sync_copy(acc.at[pl.ds(row0, ROWS_PER_TILE)],
                    out_hbm.at[cid, pl.ds(row0, ROWS_PER_TILE)])


def _sc_aggregate(y, eext, src_2d, dst_2d):
    mesh = plsc.VectorSubcoreMesh(core_axis_name="c", subcore_axis_name="s",
                                  num_cores=NC, num_subcores=NS)
    f = functools.partial(
        pl.kernel,
        out_type=jax.ShapeDtypeStruct((NC, N_PAD, EW), jnp.float32),
        mesh=mesh,
        scratch_types=[
            pltpu.VMEM((C0, CH), jnp.int32),
            pltpu.VMEM((C0, CH), jnp.int32),
            pltpu.VMEM((2, CH, YWB), jnp.bfloat16),
            pltpu.VMEM((2, CH, EW), jnp.float32),
            pltpu.VMEM((CH, EW), jnp.float32),
            pltpu.VMEM_SHARED((N_PAD, EW), jnp.float32),
            pltpu.SemaphoreType.DMA,
            pltpu.SemaphoreType.DMA,
            pltpu.SemaphoreType.DMA,
            pltpu.SemaphoreType.DMA,
        ],
        compiler_params=pltpu.CompilerParams(use_tc_tiling_on_sc=False,
                                             needs_layout_passes=False),
    )(_sc_body)
    return f(y, eext, src_2d, dst_2d)


def kernel(x, edge_index, batch, edge_attr,
           nn1_W1, nn1_b1, nn1_W2, nn1_b2, root1, bias1, bn1_w, bn1_b,
           nn2_W1, nn2_b1, nn2_W2, nn2_b2, root2, bias2, bn2_w, bn2_b,
           r1_W, r1_b, r2_W, r2_b):
    f32 = jnp.float32
    pad = E_PAD - N_EDGES
    src_p = jnp.concatenate([edge_index[0], jnp.zeros((pad,), jnp.int32)])
    dst_p = jnp.concatenate([edge_index[1], jnp.zeros((pad,), jnp.int32)])
    ea_p = jnp.concatenate([edge_attr, jnp.zeros((pad, EDIM_F), f32)], axis=0)

    perm = []
    for k in range(9):
        for i in range(HID_F):
            perm.extend([32 * k + i, 32 * k + HID_F + i])
    perm = jnp.array(perm, jnp.int32)

    def build_w2p(W2, b2, in_c):
        w = jnp.concatenate(
            [W2.reshape(HID_F, in_c, HID_F).transpose(1, 0, 2)
             .reshape(in_c, HID_F * HID_F),
             b2.reshape(in_c, HID_F),
             jnp.zeros((in_c, YWB - YW), jnp.float32)], axis=1)
        return w[:, perm]

    w2p1 = build_w2p(nn1_W2, nn1_b2, IN_F)
    w2p2 = build_w2p(nn2_W2, nn2_b2, HID_F)

    src_2d = src_p.reshape(E_PAD // CH, CH)
    dst_2d = dst_p.reshape(E_PAD // CH, CH)

    e1e, e2e = _edge_mlp(ea_p, nn1_W1, nn1_b1.reshape(1, -1),
                         nn2_W1, nn2_b1.reshape(1, -1))
    y1 = _matmul(x, w2p1, jnp.bfloat16)
    acc1 = _sc_aggregate(y1, e1e, src_2d, dst_2d)
    h1, y2 = _combine1(acc1, x, root1, bias1.reshape(1, -1),
                       bn1_w.reshape(1, -1), bn1_b.reshape(1, -1), w2p2)
    acc2 = _sc_aggregate(y2, e2e, src_2d, dst_2d)
    return _combine2(acc2, h1, root2, bias2.reshape(1, -1),
                     bn2_w.reshape(1, -1), bn2_b.reshape(1, -1),
                     batch.reshape(1, -1), r1_W, r1_b.reshape(1, -1),
                     r2_W, r2_b.reshape(1, -1))
